# baseline (speedup 1.0000x reference)
import functools

import jax
import jax.numpy as jnp
import numpy as np
from jax import lax
from jax.experimental import pallas as pl
from jax.experimental.pallas import tpu as pltpu
from jax.experimental.pallas import tpu_sc as plsc

_THR = np.float32(0.2)
_EPS = np.float32(1e-9)


def _logits_body(x_ref, wgt_ref, mean_ref, std_ref, xh_ref, lt_ref):
    x = x_ref[...]
    xh = (x - mean_ref[...]) / std_ref[...]
    xh_ref[...] = xh.astype(jnp.bfloat16)
    lt = lax.dot_general(wgt_ref[...], xh, (((1,), (1,)), ((), ())),
                         preferred_element_type=jnp.float32)
    pad = lt_ref.shape[0] - lt.shape[0]
    lt_ref[...] = jnp.concatenate(
        [lt, jnp.zeros((pad, lt.shape[1]), jnp.float32)], axis=0)


def _sc_router_body(eo, ei, b, lt_hbm, uo_hbm, ui_hbm, w16_hbm,
                    lt_v, uo_v, ui_v, w16_v):
    nc = 2
    wid = lax.axis_index("s") * nc + lax.axis_index("c")
    base = wid * 16
    for r in range(eo + eo * ei):
        pltpu.sync_copy(lt_hbm.at[pl.ds(r * b + base, 16)], lt_v.at[r])
    pltpu.sync_copy(uo_hbm.at[pl.ds(base, 16)], uo_v)
    for e in range(eo):
        pltpu.sync_copy(ui_hbm.at[pl.ds(e * b + base, 16)], ui_v.at[e])

    def softmax4(rows):
        m = jnp.maximum(jnp.maximum(rows[0], rows[1]),
                        jnp.maximum(rows[2], rows[3]))
        ex = [jnp.exp(r - m) for r in rows]
        s = ex[0] + ex[1] + ex[2] + ex[3]
        return [e / s for e in ex]

    def top1(ps):
        g = jnp.maximum(jnp.maximum(ps[0], ps[1]),
                        jnp.maximum(ps[2], ps[3]))
        o = jnp.where(ps[0] >= g, 0,
                      jnp.where(ps[1] >= g, 1,
                                jnp.where(ps[2] >= g, 2, 3)))
        return g, o.astype(jnp.int32)

    po = softmax4([lt_v[e, :] for e in range(eo)])
    g1, o1 = top1(po)
    po2 = [jnp.where(o1 == e, np.float32(-1.0), po[e]) for e in range(eo)]
    g2, o2 = top1(po2)
    den = g1 + g2 + _EPS
    go1 = g1 / den
    go2 = g2 / den
    keep2 = uo_v[...] < go2 / _THR

    qs = [softmax4([lt_v[eo + ei * e + g, :] for g in range(ei)])
          for e in range(eo)]
    zero = jnp.zeros((16,), jnp.float32)
    qb = [zero] * ei
    qc = [zero] * ei
    ub = zero
    for e in range(eo):
        sel1 = o1 == e
        sel2 = o2 == e
        for g in range(ei):
            qb[g] = jnp.where(sel1, qs[e][g], qb[g])
            qc[g] = jnp.where(sel2, qs[e][g], qc[g])
        ub = jnp.where(sel1, ui_v[e, :], ub)

    q1, i1 = top1(qb)
    qb2 = [jnp.where(i1 == g, np.float32(-1.0), qb[g]) for g in range(ei)]
    q2, i2 = top1(qb2)
    deni = q1 + q2 + _EPS
    gi1 = q1 / deni
    gi2 = q2 / deni
    keepi = ub < gi2 / _THR

    qm, j1 = top1(qc)

    pk1 = o1 * ei + i1
    pk2 = o1 * ei + i2
    pk3 = o2 * ei + j1
    wv1 = go1 * gi1
    wv2 = jnp.where(keepi, go1 * gi2, np.float32(0.0))
    wv3 = jnp.where(keep2, go2 * (qm / (qm + _EPS)), np.float32(0.0))

    zf = np.float32(0.0)
    for c in range(eo * ei):
        val = (jnp.where(pk1 == c, wv1, zf)
               + jnp.where(pk2 == c, wv2, zf)
               + jnp.where(pk3 == c, wv3, zf))
        w16_v[c, :] = val
        pltpu.sync_copy(w16_v.at[c], w16_hbm.at[pl.ds(c * b + base, 16)])


def _ffn_body(np_, nh, x_ref, xh_ref, w16t_ref, eye_ref, w1_ref, w2_ref,
              ostd_ref, omean_ref, out_ref, w16_ref):
    p = pl.program_id(0)
    h = pl.program_id(1)

    @pl.when((p == 0) & (h == 0))
    def _init():
        w16_ref[...] = lax.dot_general(
            eye_ref[...], w16t_ref[...], (((1,), (1,)), ((), ())),
            preferred_element_type=jnp.float32)
        out_ref[...] = jnp.zeros_like(out_ref)

    w1b = w1_ref[0].astype(jnp.bfloat16)
    hid = jnp.dot(xh_ref[...], w1b, preferred_element_type=jnp.float32)
    hid = jnp.maximum(hid, 0.0)
    cp = jax.lax.broadcasted_iota(jnp.int32, w16_ref.shape, 1)
    wcol = jnp.sum(jnp.where(cp == p, w16_ref[...], 0.0), axis=1,
                   keepdims=True)
    hid = (hid * wcol).astype(jnp.bfloat16)
    w2b = w2_ref[0].astype(jnp.bfloat16)
    out_ref[...] += jnp.dot(hid, w2b, preferred_element_type=jnp.float32)

    @pl.when((p == np_ - 1) & (h == nh - 1))
    def _fin():
        out_ref[...] = (x_ref[...] + out_ref[...] * ostd_ref[...]
                        + omean_ref[...])


def kernel(x, w_gate_outer, w_gate_inner, w1, w2, input_mean, input_std,
           output_mean, output_std):
    B, D = x.shape
    EO = w_gate_outer.shape[-1]
    EI = w_gate_inner.shape[-1]
    H = w1.shape[-1]
    NP = EO * EI
    HB = 2048
    NH = H // HB

    k1, k2 = jax.random.split(jax.random.key(42))
    u_out = jax.random.uniform(k1, (B, 1), dtype=jnp.float32)[:, 0]
    u_in = jax.random.uniform(k2, (EO, B, EI), dtype=jnp.float32)[:, :, 0]

    wgt = jnp.transpose(jnp.concatenate(
        [w_gate_outer,
         jnp.transpose(w_gate_inner, (1, 0, 2)).reshape(D, EO * EI)],
        axis=1))

    xh, lt = pl.pallas_call(
        _logits_body,
        out_shape=(jax.ShapeDtypeStruct((B, D), jnp.bfloat16),
                   jax.ShapeDtypeStruct((32, B), jnp.float32)),
    )(x, wgt, input_mean.reshape(1, D), input_std.reshape(1, D))

    mesh = plsc.VectorSubcoreMesh(core_axis_name="c", subcore_axis_name="s")
    w16t = pl.kernel(
        functools.partial(_sc_router_body, EO, EI, B),
        out_type=jax.ShapeDtypeStruct((NP * B,), jnp.float32),
        mesh=mesh,
        scratch_types=[
            pltpu.VMEM((32, 16), jnp.float32),
            pltpu.VMEM((16,), jnp.float32),
            pltpu.VMEM((EO, 16), jnp.float32),
            pltpu.VMEM((NP, 16), jnp.float32),
        ],
    )(lt[:EO + EO * EI].reshape(-1), u_out, u_in.reshape(-1))
    w16t = w16t.reshape(NP, B)

    w1f = w1.reshape(NP, D, H)
    w2f = w2.reshape(NP, H, D)

    out = pl.pallas_call(
        functools.partial(_ffn_body, NP, NH),
        grid=(NP, NH),
        in_specs=[
            pl.BlockSpec((B, D), lambda p, h: (0, 0)),
            pl.BlockSpec((B, D), lambda p, h: (0, 0)),
            pl.BlockSpec((NP, B), lambda p, h: (0, 0)),
            pl.BlockSpec((B, B), lambda p, h: (0, 0)),
            pl.BlockSpec((1, D, HB), lambda p, h: (p, 0, h)),
            pl.BlockSpec((1, HB, D), lambda p, h: (p, h, 0)),
            pl.BlockSpec((1, D), lambda p, h: (0, 0)),
            pl.BlockSpec((1, D), lambda p, h: (0, 0)),
        ],
        out_specs=pl.BlockSpec((B, D), lambda p, h: (0, 0)),
        out_shape=jax.ShapeDtypeStruct((B, D), jnp.float32),
        scratch_shapes=[pltpu.VMEM((B, NP), jnp.float32)],
        compiler_params=pltpu.CompilerParams(
            dimension_semantics=("arbitrary", "arbitrary")),
    )(x, xh, w16t, jnp.eye(B, dtype=jnp.float32), w1f, w2f,
      output_std.reshape(1, D), output_mean.reshape(1, D))
    return out


# speedup vs baseline: 1.1234x; 1.1234x over previous
import functools

import jax
import jax.numpy as jnp
from jax.experimental import pallas as pl
from jax.experimental.pallas import tpu as pltpu

import numpy as np

_THR = np.float32(0.2)
_EPS = np.float32(1e-9)


def _top2(p):
    c = jax.lax.broadcasted_iota(jnp.int32, p.shape, 1)
    m1 = jnp.max(p, axis=1, keepdims=True)
    i1 = jnp.min(jnp.where(p >= m1, c, p.shape[1]), axis=1, keepdims=True)
    p2 = jnp.where(c == i1, jnp.float32(-1.0), p)
    m2 = jnp.max(p2, axis=1, keepdims=True)
    i2 = jnp.min(jnp.where(p2 >= m2, c, p.shape[1]), axis=1, keepdims=True)
    return m1, i1, m2, i2


def _softmax(l):
    e = jnp.exp(l - jnp.max(l, axis=1, keepdims=True))
    return e / jnp.sum(e, axis=1, keepdims=True)


def _router_body(eo, ei, x_ref, wg_ref, uo_ref, ui_ref, mean_ref, std_ref,
                 xh_ref, w16_ref):
    x = x_ref[...]
    xh = (x - mean_ref[...]) / std_ref[...]
    xh_ref[...] = xh.astype(jnp.bfloat16)
    logits = jnp.dot(xh, wg_ref[...], preferred_element_type=jnp.float32)

    po = _softmax(logits[:, 0:eo])
    g1, o1, g2, o2 = _top2(po)
    den = g1 + g2 + _EPS
    go1 = g1 / den
    go2 = g2 / den
    keep2 = (uo_ref[...] < go2 / _THR).astype(jnp.float32)

    qs = [_softmax(logits[:, eo + ei * e: eo + ei * (e + 1)]) for e in range(eo)]
    zero = jnp.zeros_like(qs[0])
    qb = zero
    qc = zero
    ub = jnp.zeros_like(g1)
    for e in range(eo):
        qb = qb + jnp.where(o1 == e, qs[e], 0.0)
        qc = qc + jnp.where(o2 == e, qs[e], 0.0)
        ub = ub + jnp.where(o1 == e, ui_ref[:, e:e + 1], 0.0)

    q1, i1, q2, i2 = _top2(qb)
    deni = q1 + q2 + _EPS
    gi1 = q1 / deni
    gi2 = q2 / deni
    keep_i2 = (ub < gi2 / _THR).astype(jnp.float32)

    qm, j1, _, _ = _top2(qc)
    w3 = go2 * (qm / (qm + _EPS)) * keep2

    cp = jax.lax.broadcasted_iota(jnp.int32, (x.shape[0], eo * ei), 1)
    w16 = (jnp.where(cp == o1 * ei + i1, go1 * gi1, 0.0)
           + jnp.where(cp == o1 * ei + i2, keep_i2 * go1 * gi2, 0.0)
           + jnp.where(cp == o2 * ei + j1, w3, 0.0))
    w16_ref[...] = w16


def _ffn_body(eo, ei, np_, nh, x_ref, wg_ref, uo_ref, ui_ref, mean_ref,
              std_ref, w1_ref, w2_ref, ostd_ref, omean_ref, out_ref,
              xh_ref, w16_ref):
    p = pl.program_id(0)
    h = pl.program_id(1)

    @pl.when((p == 0) & (h == 0))
    def _init():
        _router_body(eo, ei, x_ref, wg_ref, uo_ref, ui_ref, mean_ref,
                     std_ref, xh_ref, w16_ref)
        out_ref[...] = jnp.zeros_like(out_ref)

    w1b = w1_ref[0].astype(jnp.bfloat16)
    hid = jnp.dot(xh_ref[...], w1b, preferred_element_type=jnp.float32)
    hid = jnp.maximum(hid, 0.0)
    cp = jax.lax.broadcasted_iota(jnp.int32, w16_ref.shape, 1)
    wcol = jnp.sum(jnp.where(cp == p, w16_ref[...], 0.0), axis=1, keepdims=True)
    hid = (hid * wcol).astype(jnp.bfloat16)
    w2b = w2_ref[0].astype(jnp.bfloat16)
    out_ref[...] += jnp.dot(hid, w2b, preferred_element_type=jnp.float32)

    @pl.when((p == np_ - 1) & (h == nh - 1))
    def _fin():
        out_ref[...] = (x_ref[...] + out_ref[...] * ostd_ref[...]
                        + omean_ref[...])


def kernel(x, w_gate_outer, w_gate_inner, w1, w2, input_mean, input_std,
           output_mean, output_std):
    B, D = x.shape
    EO = w_gate_outer.shape[-1]
    EI = w_gate_inner.shape[-1]
    H = w1.shape[-1]
    NP = EO * EI
    HB = 2048
    NH = H // HB

    k1, k2 = jax.random.split(jax.random.key(42))
    u_out = jax.random.uniform(k1, (B, 1), dtype=jnp.float32)
    u_in = jnp.transpose(jax.random.uniform(k2, (EO, B, EI),
                                            dtype=jnp.float32)[:, :, 0])

    wg = jnp.concatenate(
        [w_gate_outer,
         jnp.transpose(w_gate_inner, (1, 0, 2)).reshape(D, EO * EI)], axis=1)

    w1f = w1.reshape(NP, D, H)
    w2f = w2.reshape(NP, H, D)

    out = pl.pallas_call(
        functools.partial(_ffn_body, EO, EI, NP, NH),
        grid=(NP, NH),
        in_specs=[
            pl.BlockSpec((B, D), lambda p, h: (0, 0)),
            pl.BlockSpec(wg.shape, lambda p, h: (0, 0)),
            pl.BlockSpec((B, 1), lambda p, h: (0, 0)),
            pl.BlockSpec((B, EO), lambda p, h: (0, 0)),
            pl.BlockSpec((1, D), lambda p, h: (0, 0)),
            pl.BlockSpec((1, D), lambda p, h: (0, 0)),
            pl.BlockSpec((1, D, HB), lambda p, h: (p, 0, h)),
            pl.BlockSpec((1, HB, D), lambda p, h: (p, h, 0)),
            pl.BlockSpec((1, D), lambda p, h: (0, 0)),
            pl.BlockSpec((1, D), lambda p, h: (0, 0)),
        ],
        out_specs=pl.BlockSpec((B, D), lambda p, h: (0, 0)),
        out_shape=jax.ShapeDtypeStruct((B, D), jnp.float32),
        scratch_shapes=[
            pltpu.VMEM((B, D), jnp.bfloat16),
            pltpu.VMEM((B, NP), jnp.float32),
        ],
        compiler_params=pltpu.CompilerParams(
            dimension_semantics=("arbitrary", "arbitrary"),
            vmem_limit_bytes=80 * 1024 * 1024),
    )(x, wg, u_out, u_in, input_mean.reshape(1, D), input_std.reshape(1, D),
      w1f, w2f, output_std.reshape(1, D), output_mean.reshape(1, D))
    return out
